# baseline (speedup 1.0000x reference)
import jax
import jax.numpy as jnp
from jax import lax
from jax.experimental import pallas as pl
from jax.experimental.pallas import tpu as pltpu
from jax.experimental.pallas import tpu_sc as plsc

N = 10000
E = 320000
T = 256
D = 128
H = 4
DH = 32
B = 4
OUT = 4

NT = 400
CE = 128
NCHB = (E // CE) // 16
NCHR = (E // CE) % 16
NPAD = 10112
ZR = NPAD // 16
F32 = jnp.float32
I32 = jnp.int32

_GDN = lax.GatherDimensionNumbers(
    offset_dims=(), collapsed_slice_dims=(0,), start_index_map=(0,))


def _rot(v, idx):
    return lax.gather(v, idx[:, None], _GDN, (1,),
                      mode=lax.GatherScatterMode.PROMISE_IN_BOUNDS)


def _tc_body(x_ref, w_ref, b_ref, q_ref, k_ref, v_ref, xs_ref):
    j = pl.program_id(1)
    x = x_ref[0]
    y = jnp.dot(x, w_ref[...], preferred_element_type=F32) + b_ref[0]
    q_ref[0] = y[:, :D]
    k_ref[0] = y[:, D:2 * D]
    v_ref[0] = y[:, 2 * D:]

    @pl.when(j == 0)
    def _():
        xs_ref[0, 0] = jnp.zeros((T,), F32)

    xs_ref[0, 0] += jnp.sum(x, axis=0)


def _tc_project(X, Wfold, bfold):
    grid = (B, N // NT)
    return pl.pallas_call(
        _tc_body,
        grid=grid,
        in_specs=[
            pl.BlockSpec((1, NT, T), lambda b, j: (b, j, 0)),
            pl.BlockSpec((T, 3 * D), lambda b, j: (0, 0)),
            pl.BlockSpec((1, 3 * D), lambda b, j: (0, 0)),
        ],
        out_specs=[
            pl.BlockSpec((1, NT, D), lambda b, j: (b, j, 0)),
            pl.BlockSpec((1, NT, D), lambda b, j: (b, j, 0)),
            pl.BlockSpec((1, NT, D), lambda b, j: (b, j, 0)),
            pl.BlockSpec((1, 1, T), lambda b, j: (b, 0, 0)),
        ],
        out_shape=[
            jax.ShapeDtypeStruct((B, N, D), F32),
            jax.ShapeDtypeStruct((B, N, D), F32),
            jax.ShapeDtypeStruct((B, N, D), F32),
            jax.ShapeDtypeStruct((B, 1, T), F32),
        ],
    )(X, Wfold, bfold)


TBL = 81920
SLC = TBL // 16
RB = 1280


CH = CE // 2


def _sc_body(q_hbm, k_hbm, v_hbm, ei_hbm,
             acc_hbm, ex_hbm, stag_hbm, glob_hbm,
             qbufA, kbufA, qbufB, kbufB, tbl, tmp2, expacked,
             edb, ivqA, ivkA, ivqB, ivkB, dsth0, dsth1, accst,
             accstage_sh, semQA, semKA, semQB, semKB):
    cid = lax.axis_index("c")
    sid = lax.axis_index("s")
    iota16 = lax.iota(I32, 16)
    zf = jnp.zeros((16,), F32)
    nch = NCHB + jnp.where(sid < NCHR, 1, 0)
    tbase = (sid * NCHB + jnp.minimum(sid, NCHR)) * CE
    sslc = pl.multiple_of(sid * SLC, 128)
    stbl = pl.multiple_of(sid * TBL, 128)

    def load_edb(off):
        off = pl.multiple_of(off, CE)
        pltpu.sync_copy(ei_hbm.at[:, pl.ds(off, CE)], edb)

    def build_qk(b, half, ivq, ivk, dsth):
        bN = b * N
        for i in range(CH // 16):
            sl = pl.ds(i * 16, 16)
            se = pl.ds(half * CH + i * 16, 16)
            dv = edb[1, se]
            ivq[sl] = dv + bN
            ivk[sl] = edb[0, se] + bN
            dsth[sl] = dv
        return (pltpu.async_copy(q_hbm.at[ivq], qbufA if ivq is ivqA else qbufB,
                                 semQA if ivq is ivqA else semQB),
                pltpu.async_copy(k_hbm.at[ivk], kbufA if ivk is ivkA else kbufB,
                                 semKA if ivk is ivkA else semKB))

    def build_v(b, half, ivq, dsth):
        bN = b * N
        for i in range(CH // 16):
            sl = pl.ds(i * 16, 16)
            se = pl.ds(half * CH + i * 16, 16)
            ivq[sl] = edb[0, se] + bN
            dsth[sl] = edb[1, se]
        return pltpu.async_copy(v_hbm.at[ivq], qbufA if ivq is ivqA else qbufB,
                                semQA if ivq is ivqA else semQB)

    def p1_compute(qb, kb, dsth, half, _g_unused=None):
        def gbody(g, _):
            i16 = lax.iota(I32, 16)
            rr = [(i16 + 8) & 15, (i16 + 4) & 15, (i16 + 2) & 15, (i16 + 1) & 15]
            dvec = dsth[pl.ds(pl.multiple_of(g * 16, 16), 16)]
            for j in range(16):
                er = g * 16 + j
                row = jnp.zeros((16,), F32)
                for h in range(H):
                    pr = (qb[er, pl.ds(2 * h * 16, 16)] * kb[er, pl.ds(2 * h * 16, 16)]
                          + qb[er, pl.ds((2 * h + 1) * 16, 16)]
                          * kb[er, pl.ds((2 * h + 1) * 16, 16)])
                    for r in rr:
                        pr = pr + _rot(pr, r)
                    row = row + jnp.where(i16 == h, pr, 0.0)
                exr = jnp.where(i16 < H, jnp.exp(row), 0.0)
                dn = dvec[j]
                o8 = pl.ds(pl.multiple_of(dn * 8, 8), 16)
                tbl[o8] += exr
                expacked[half * 8 + g * 2 + j // 8, pl.ds((j & 7) * 16, 16)] = exr
            return 0
        lax.fori_loop(0, CH // 16, gbody, 0)

    def p2_compute(qb, dsth, half, acc):
        def gbody(g, a):
            dvec = dsth[pl.ds(pl.multiple_of(g * 16, 16), 16)]
            out = list(a)
            for j in range(16):
                er = g * 16 + j
                dn = dvec[j]
                dnm = tbl[pl.ds(pl.multiple_of(dn * 8, 8), 16)]
                exr = expacked[half * 8 + g * 2 + j // 8, pl.ds((j & 7) * 16, 16)]
                w = exr / dnm
                for h in range(H):
                    wh = w[h]
                    out[2 * h] = out[2 * h] + qb[er, pl.ds(2 * h * 16, 16)] * wh
                    out[2 * h + 1] = (out[2 * h + 1]
                                      + qb[er, pl.ds((2 * h + 1) * 16, 16)] * wh)
            return tuple(out)
        return lax.fori_loop(0, CH // 16, gbody, acc)

    def batch_body(ib, _):
        b = cid * 2 + ib

        def ztbl(i, _):
            tbl[pl.ds(pl.multiple_of(i * 16, 16), 16)] = zf
            return 0
        lax.fori_loop(0, TBL // 16, ztbl, 0)

        load_edb(tbase)
        cqA, ckA = build_qk(b, 0, ivqA, ivkA, dsth0)

        def p1_loop(c, _):
            off = tbase + c * CE
            off8 = pl.multiple_of(off // 8, CE // 8)
            cqB, ckB = build_qk(b, 1, ivqB, ivkB, dsth1)

            @pl.when(c + 1 < nch)
            def _():
                load_edb(off + CE)
            pltpu.make_async_copy(q_hbm.at[ivqA], qbufA, semQA).wait()
            pltpu.make_async_copy(k_hbm.at[ivkA], kbufA, semKA).wait()
            p1_compute(qbufA, kbufA, dsth0, 0)

            @pl.when(c + 1 < nch)
            def _():
                build_qk(b, 0, ivqA, ivkA, dsth0)
            pltpu.make_async_copy(q_hbm.at[ivqB], qbufB, semQB).wait()
            pltpu.make_async_copy(k_hbm.at[ivkB], kbufB, semKB).wait()
            p1_compute(qbufB, kbufB, dsth1, 1)
            pltpu.sync_copy(expacked, ex_hbm.at[cid].at[pl.ds(off8, CE // 8)])
            return 0

        lax.fori_loop(0, nch, p1_loop, 0)

        pltpu.sync_copy(tbl, stag_hbm.at[cid].at[pl.ds(stbl, TBL)])
        plsc.subcore_barrier()
        for t in range(16):
            if t == 0:
                continue
            ot = (sid + t) % 16

            def rsub(q, _):
                qo = pl.multiple_of(q * RB, RB)
                pltpu.sync_copy(
                    stag_hbm.at[cid].at[pl.ds(ot * TBL + sslc + qo, RB)], tmp2)

                def radd2(v, _):
                    svo = pl.multiple_of(q * RB + v * 16, 16)
                    tbl[pl.ds(sslc + svo, 16)] += tmp2[pl.ds(pl.multiple_of(v * 16, 16), 16)]
                    return 0
                lax.fori_loop(0, RB // 16, radd2, 0)
                return 0
            lax.fori_loop(0, SLC // RB, rsub, 0)
        pltpu.sync_copy(tbl.at[pl.ds(sslc, SLC)], glob_hbm.at[cid].at[pl.ds(sslc, SLC)])
        plsc.subcore_barrier()
        pltpu.sync_copy(glob_hbm.at[cid], tbl)

        load_edb(tbase)
        cvA = build_v(b, 0, ivqA, dsth0)

        def p2_loop(c, acc):
            off = tbase + c * CE
            off8 = pl.multiple_of(off // 8, CE // 8)
            cvB = build_v(b, 1, ivqB, dsth1)
            pltpu.sync_copy(ex_hbm.at[cid].at[pl.ds(off8, CE // 8)], expacked)

            @pl.when(c + 1 < nch)
            def _():
                load_edb(off + CE)
            pltpu.make_async_copy(v_hbm.at[ivqA], qbufA, semQA).wait()
            acc = p2_compute(qbufA, dsth0, 0, acc)

            @pl.when(c + 1 < nch)
            def _():
                build_v(b, 0, ivqA, dsth0)
            pltpu.make_async_copy(v_hbm.at[ivqB], qbufB, semQB).wait()
            acc = p2_compute(qbufB, dsth1, 1, acc)
            return acc

        acc = lax.fori_loop(0, nch, p2_loop, (zf,) * 8)

        for i in range(8):
            accst[0, pl.ds(i * 16, 16)] = acc[i]
        for r in range(1, 8):
            for i in range(8):
                accst[r, pl.ds(i * 16, 16)] = zf
        pltpu.sync_copy(accst, accstage_sh.at[pl.ds(pl.multiple_of(sid * 8, 8), 8)])
        plsc.subcore_barrier()

        @pl.when(sid == 0)
        def _():
            pltpu.sync_copy(accstage_sh.at[pl.ds(0, 64)], qbufA)
            pltpu.sync_copy(accstage_sh.at[pl.ds(64, 64)], qbufB)
            for i in range(8):
                r = jnp.zeros((16,), F32)
                for t in range(8):
                    r = (r + qbufA[t * 8, pl.ds(i * 16, 16)]
                         + qbufB[t * 8, pl.ds(i * 16, 16)])
                accst[0, pl.ds(i * 16, 16)] = r
            pltpu.sync_copy(accst, acc_hbm.at[pl.ds(pl.multiple_of(b * 8, 8), 8)])

        plsc.subcore_barrier()
        return 0

    lax.fori_loop(0, 2, batch_body, 0)


def _sc_edge_attention(Qf, Kf, Vf, ei):
    mesh = plsc.VectorSubcoreMesh(core_axis_name="c", subcore_axis_name="s")
    f = pl.kernel(
        _sc_body,
        mesh=mesh,
        out_type=(
            jax.ShapeDtypeStruct((B * 8, D), F32),
            jax.ShapeDtypeStruct((2, E // 8, 128), F32),
            jax.ShapeDtypeStruct((2, 16 * TBL), F32),
            jax.ShapeDtypeStruct((2, TBL), F32),
        ),
        scratch_types=[
            pltpu.VMEM((CH, D), F32),
            pltpu.VMEM((CH, D), F32),
            pltpu.VMEM((CH, D), F32),
            pltpu.VMEM((CH, D), F32),
            pltpu.VMEM((TBL,), F32),
            pltpu.VMEM((RB,), F32),
            pltpu.VMEM((CE // 8, 128), F32),
            pltpu.VMEM((2, CE), I32),
            pltpu.VMEM((CH,), I32),
            pltpu.VMEM((CH,), I32),
            pltpu.VMEM((CH,), I32),
            pltpu.VMEM((CH,), I32),
            pltpu.VMEM((CH,), I32),
            pltpu.VMEM((CH,), I32),
            pltpu.VMEM((8, 128), F32),
            pltpu.VMEM_SHARED((128, 128), F32),
            pltpu.SemaphoreType.DMA,
            pltpu.SemaphoreType.DMA,
            pltpu.SemaphoreType.DMA,
            pltpu.SemaphoreType.DMA,
        ],
    )
    return f(Qf, Kf, Vf, ei)


def kernel(X, edge_index, W_in, b_in, Wq, bq, Wk, bk, Wv, bv,
           Wskip, bskip, Wread, bread):
    scale = 1.0 / jnp.sqrt(jnp.float32(DH))
    Wfold = jnp.concatenate(
        [W_in @ Wq * scale, W_in @ Wk, W_in @ Wv], axis=1)
    bfold = jnp.concatenate(
        [(b_in @ Wq + bq) * scale, b_in @ Wk + bk, b_in @ Wv + bv])[None, :]

    Q, K, V, Xs = _tc_project(X, Wfold, bfold)
    Qf = Q.reshape(B * N, D)
    Kf = K.reshape(B * N, D)
    Vf = V.reshape(B * N, D)

    acc8 = _sc_edge_attention(Qf, Kf, Vf, edge_index)[0]
    acc = acc8.reshape(B, 8, D)[:, 0, :]

    skip = ((Xs[:, 0, :] / N) @ W_in + b_in) @ Wskip + bskip
    graph_rep = acc / N + skip
    return graph_rep @ Wread + bread


# speedup vs baseline: 1.1590x; 1.1590x over previous
import jax
import jax.numpy as jnp
from jax import lax
from jax.experimental import pallas as pl
from jax.experimental.pallas import tpu as pltpu
from jax.experimental.pallas import tpu_sc as plsc

N = 10000
E = 320000
T = 256
D = 128
H = 4
DH = 32
B = 4
OUT = 4

NT = 400
CE = 128
NCHB = (E // CE) // 16
NCHR = (E // CE) % 16
NPAD = 10112
ZR = NPAD // 16
F32 = jnp.float32
I32 = jnp.int32

_GDN = lax.GatherDimensionNumbers(
    offset_dims=(), collapsed_slice_dims=(0,), start_index_map=(0,))


def _rot(v, idx):
    return lax.gather(v, idx[:, None], _GDN, (1,),
                      mode=lax.GatherScatterMode.PROMISE_IN_BOUNDS)


def _tc_body(x_ref, w_ref, b_ref, q_ref, k_ref, v_ref, xs_ref):
    j = pl.program_id(1)
    x = x_ref[0]
    y = jnp.dot(x, w_ref[...], preferred_element_type=F32) + b_ref[0]
    q_ref[0] = y[:, :D]
    k_ref[0] = y[:, D:2 * D]
    v_ref[0] = y[:, 2 * D:]

    @pl.when(j == 0)
    def _():
        xs_ref[0, 0] = jnp.zeros((T,), F32)

    xs_ref[0, 0] += jnp.sum(x, axis=0)


def _tc_project(X, Wfold, bfold):
    grid = (B, N // NT)
    return pl.pallas_call(
        _tc_body,
        grid=grid,
        in_specs=[
            pl.BlockSpec((1, NT, T), lambda b, j: (b, j, 0)),
            pl.BlockSpec((T, 3 * D), lambda b, j: (0, 0)),
            pl.BlockSpec((1, 3 * D), lambda b, j: (0, 0)),
        ],
        out_specs=[
            pl.BlockSpec((1, NT, D), lambda b, j: (b, j, 0)),
            pl.BlockSpec((1, NT, D), lambda b, j: (b, j, 0)),
            pl.BlockSpec((1, NT, D), lambda b, j: (b, j, 0)),
            pl.BlockSpec((1, 1, T), lambda b, j: (b, 0, 0)),
        ],
        out_shape=[
            jax.ShapeDtypeStruct((B, N, D), F32),
            jax.ShapeDtypeStruct((B, N, D), F32),
            jax.ShapeDtypeStruct((B, N, D), F32),
            jax.ShapeDtypeStruct((B, 1, T), F32),
        ],
    )(X, Wfold, bfold)


TBL = 81920
SLC = TBL // 16
RB = 1280


CH = CE // 2


def _sc_body(q_hbm, k_hbm, v_hbm, ei_hbm,
             acc_hbm, ex_hbm, stag_hbm, glob_hbm,
             qbufA, kbufA, qbufB, kbufB, tbl, tmp2, expacked,
             edb, ivqA, ivkA, ivqB, ivkB, dsth0, dsth1, accst,
             accstage_sh, semQA, semKA, semQB, semKB, semE):
    cid = lax.axis_index("c")
    sid = lax.axis_index("s")
    iota16 = lax.iota(I32, 16)
    zf = jnp.zeros((16,), F32)
    nch = NCHB + jnp.where(sid < NCHR, 1, 0)
    tbase = (sid * NCHB + jnp.minimum(sid, NCHR)) * CE
    sslc = pl.multiple_of(sid * SLC, 128)
    stbl = pl.multiple_of(sid * TBL, 128)

    def load_edb(off):
        off = pl.multiple_of(off, CE)
        pltpu.sync_copy(ei_hbm.at[:, pl.ds(off, CE)], edb)

    def load_edb_async(off):
        off = pl.multiple_of(off, CE)
        pltpu.async_copy(ei_hbm.at[:, pl.ds(off, CE)], edb, semE)

    def wait_edb(off):
        off = pl.multiple_of(off, CE)
        pltpu.make_async_copy(ei_hbm.at[:, pl.ds(off, CE)], edb, semE).wait()

    def build_qk(b, half, ivq, ivk, dsth):
        bN = b * N
        for i in range(CH // 16):
            sl = pl.ds(i * 16, 16)
            se = pl.ds(half * CH + i * 16, 16)
            dv = edb[1, se]
            ivq[sl] = dv + bN
            ivk[sl] = edb[0, se] + bN
            dsth[sl] = dv
        return (pltpu.async_copy(q_hbm.at[ivq], qbufA if ivq is ivqA else qbufB,
                                 semQA if ivq is ivqA else semQB),
                pltpu.async_copy(k_hbm.at[ivk], kbufA if ivk is ivkA else kbufB,
                                 semKA if ivk is ivkA else semKB))

    def build_v(b, half, ivq, dsth):
        bN = b * N
        for i in range(CH // 16):
            sl = pl.ds(i * 16, 16)
            se = pl.ds(half * CH + i * 16, 16)
            ivq[sl] = edb[0, se] + bN
            dsth[sl] = edb[1, se]
        return pltpu.async_copy(v_hbm.at[ivq], qbufA if ivq is ivqA else qbufB,
                                semQA if ivq is ivqA else semQB)

    def p1_compute(qb, kb, dsth, half, _g_unused=None):
        def gbody(g, _):
            i16 = lax.iota(I32, 16)
            rr = [(i16 + 8) & 15, (i16 + 4) & 15, (i16 + 2) & 15, (i16 + 1) & 15]
            dvec = dsth[pl.ds(pl.multiple_of(g * 16, 16), 16)]
            for j in range(16):
                er = g * 16 + j
                row = jnp.zeros((16,), F32)
                for h in range(H):
                    pr = (qb[er, pl.ds(2 * h * 16, 16)] * kb[er, pl.ds(2 * h * 16, 16)]
                          + qb[er, pl.ds((2 * h + 1) * 16, 16)]
                          * kb[er, pl.ds((2 * h + 1) * 16, 16)])
                    for r in rr:
                        pr = pr + _rot(pr, r)
                    row = row + jnp.where(i16 == h, pr, 0.0)
                exr = jnp.where(i16 < H, jnp.exp(row), 0.0)
                dn = dvec[j]
                o8 = pl.ds(pl.multiple_of(dn * 8, 8), 16)
                tbl[o8] += exr
                expacked[half * 8 + g * 2 + j // 8, pl.ds((j & 7) * 16, 16)] = exr
            return 0
        lax.fori_loop(0, CH // 16, gbody, 0)

    def p2_compute(qb, dsth, half, acc):
        def gbody(g, a):
            dvec = dsth[pl.ds(pl.multiple_of(g * 16, 16), 16)]
            out = list(a)
            for j in range(16):
                er = g * 16 + j
                dn = dvec[j]
                dnm = tbl[pl.ds(pl.multiple_of(dn * 8, 8), 16)]
                exr = expacked[half * 8 + g * 2 + j // 8, pl.ds((j & 7) * 16, 16)]
                w = exr / dnm
                for h in range(H):
                    wh = w[h]
                    out[2 * h] = out[2 * h] + qb[er, pl.ds(2 * h * 16, 16)] * wh
                    out[2 * h + 1] = (out[2 * h + 1]
                                      + qb[er, pl.ds((2 * h + 1) * 16, 16)] * wh)
            return tuple(out)
        return lax.fori_loop(0, CH // 16, gbody, acc)

    def batch_body(ib, _):
        b = cid * 2 + ib

        def ztbl(i, _):
            tbl[pl.ds(pl.multiple_of(i * 16, 16), 16)] = zf
            return 0
        lax.fori_loop(0, TBL // 16, ztbl, 0)

        load_edb(tbase)
        cqA, ckA = build_qk(b, 0, ivqA, ivkA, dsth0)

        def p1_loop(c, _):
            off = tbase + c * CE
            off8 = pl.multiple_of(off // 8, CE // 8)
            cqB, ckB = build_qk(b, 1, ivqB, ivkB, dsth1)

            @pl.when(c + 1 < nch)
            def _():
                load_edb_async(off + CE)
            pltpu.make_async_copy(q_hbm.at[ivqA], qbufA, semQA).wait()
            pltpu.make_async_copy(k_hbm.at[ivkA], kbufA, semKA).wait()
            p1_compute(qbufA, kbufA, dsth0, 0)

            @pl.when(c + 1 < nch)
            def _():
                wait_edb(off + CE)
                build_qk(b, 0, ivqA, ivkA, dsth0)
            pltpu.make_async_copy(q_hbm.at[ivqB], qbufB, semQB).wait()
            pltpu.make_async_copy(k_hbm.at[ivkB], kbufB, semKB).wait()
            p1_compute(qbufB, kbufB, dsth1, 1)
            pltpu.sync_copy(expacked, ex_hbm.at[cid].at[pl.ds(off8, CE // 8)])
            return 0

        lax.fori_loop(0, nch, p1_loop, 0)

        pltpu.sync_copy(tbl, stag_hbm.at[cid].at[pl.ds(stbl, TBL)])
        plsc.subcore_barrier()
        for t in range(16):
            if t == 0:
                continue
            ot = (sid + t) % 16

            def rsub(q, _):
                qo = pl.multiple_of(q * RB, RB)
                pltpu.sync_copy(
                    stag_hbm.at[cid].at[pl.ds(ot * TBL + sslc + qo, RB)], tmp2)

                def radd2(v, _):
                    svo = pl.multiple_of(q * RB + v * 16, 16)
                    tbl[pl.ds(sslc + svo, 16)] += tmp2[pl.ds(pl.multiple_of(v * 16, 16), 16)]
                    return 0
                lax.fori_loop(0, RB // 16, radd2, 0)
                return 0
            lax.fori_loop(0, SLC // RB, rsub, 0)
        pltpu.sync_copy(tbl.at[pl.ds(sslc, SLC)], glob_hbm.at[cid].at[pl.ds(sslc, SLC)])
        plsc.subcore_barrier()
        pltpu.sync_copy(glob_hbm.at[cid], tbl)

        load_edb(tbase)
        cvA = build_v(b, 0, ivqA, dsth0)

        def p2_loop(c, acc):
            off = tbase + c * CE
            off8 = pl.multiple_of(off // 8, CE // 8)
            cvB = build_v(b, 1, ivqB, dsth1)
            pltpu.sync_copy(ex_hbm.at[cid].at[pl.ds(off8, CE // 8)], expacked)

            @pl.when(c + 1 < nch)
            def _():
                load_edb_async(off + CE)
            pltpu.make_async_copy(v_hbm.at[ivqA], qbufA, semQA).wait()
            acc = p2_compute(qbufA, dsth0, 0, acc)

            @pl.when(c + 1 < nch)
            def _():
                wait_edb(off + CE)
                build_v(b, 0, ivqA, dsth0)
            pltpu.make_async_copy(v_hbm.at[ivqB], qbufB, semQB).wait()
            acc = p2_compute(qbufB, dsth1, 1, acc)
            return acc

        acc = lax.fori_loop(0, nch, p2_loop, (zf,) * 8)

        for i in range(8):
            accst[0, pl.ds(i * 16, 16)] = acc[i]
        for r in range(1, 8):
            for i in range(8):
                accst[r, pl.ds(i * 16, 16)] = zf
        pltpu.sync_copy(accst, accstage_sh.at[pl.ds(pl.multiple_of(sid * 8, 8), 8)])
        plsc.subcore_barrier()

        @pl.when(sid == 0)
        def _():
            pltpu.sync_copy(accstage_sh.at[pl.ds(0, 64)], qbufA)
            pltpu.sync_copy(accstage_sh.at[pl.ds(64, 64)], qbufB)
            for i in range(8):
                r = jnp.zeros((16,), F32)
                for t in range(8):
                    r = (r + qbufA[t * 8, pl.ds(i * 16, 16)]
                         + qbufB[t * 8, pl.ds(i * 16, 16)])
                accst[0, pl.ds(i * 16, 16)] = r
            pltpu.sync_copy(accst, acc_hbm.at[pl.ds(pl.multiple_of(b * 8, 8), 8)])

        plsc.subcore_barrier()
        return 0

    lax.fori_loop(0, 2, batch_body, 0)


def _sc_edge_attention(Qf, Kf, Vf, ei):
    mesh = plsc.VectorSubcoreMesh(core_axis_name="c", subcore_axis_name="s")
    f = pl.kernel(
        _sc_body,
        mesh=mesh,
        out_type=(
            jax.ShapeDtypeStruct((B * 8, D), F32),
            jax.ShapeDtypeStruct((2, E // 8, 128), F32),
            jax.ShapeDtypeStruct((2, 16 * TBL), F32),
            jax.ShapeDtypeStruct((2, TBL), F32),
        ),
        scratch_types=[
            pltpu.VMEM((CH, D), F32),
            pltpu.VMEM((CH, D), F32),
            pltpu.VMEM((CH, D), F32),
            pltpu.VMEM((CH, D), F32),
            pltpu.VMEM((TBL,), F32),
            pltpu.VMEM((RB,), F32),
            pltpu.VMEM((CE // 8, 128), F32),
            pltpu.VMEM((2, CE), I32),
            pltpu.VMEM((CH,), I32),
            pltpu.VMEM((CH,), I32),
            pltpu.VMEM((CH,), I32),
            pltpu.VMEM((CH,), I32),
            pltpu.VMEM((CH,), I32),
            pltpu.VMEM((CH,), I32),
            pltpu.VMEM((8, 128), F32),
            pltpu.VMEM_SHARED((128, 128), F32),
            pltpu.SemaphoreType.DMA,
            pltpu.SemaphoreType.DMA,
            pltpu.SemaphoreType.DMA,
            pltpu.SemaphoreType.DMA,
            pltpu.SemaphoreType.DMA,
        ],
    )
    return f(Qf, Kf, Vf, ei)


def kernel(X, edge_index, W_in, b_in, Wq, bq, Wk, bk, Wv, bv,
           Wskip, bskip, Wread, bread):
    scale = 1.0 / jnp.sqrt(jnp.float32(DH))
    Wfold = jnp.concatenate(
        [W_in @ Wq * scale, W_in @ Wk, W_in @ Wv], axis=1)
    bfold = jnp.concatenate(
        [(b_in @ Wq + bq) * scale, b_in @ Wk + bk, b_in @ Wv + bv])[None, :]

    Q, K, V, Xs = _tc_project(X, Wfold, bfold)
    Qf = Q.reshape(B * N, D)
    Kf = K.reshape(B * N, D)
    Vf = V.reshape(B * N, D)

    acc8 = _sc_edge_attention(Qf, Kf, Vf, edge_index)[0]
    acc = acc8.reshape(B, 8, D)[:, 0, :]

    skip = ((Xs[:, 0, :] / N) @ W_in + b_in) @ Wskip + bskip
    graph_rep = acc / N + skip
    return graph_rep @ Wread + bread
